# baseline (speedup 1.0000x reference)
import functools

import jax
import jax.numpy as jnp
from jax import lax
from jax.experimental import pallas as pl
from jax.experimental.pallas import tpu as pltpu
from jax.experimental.pallas import tpu_sc as plsc

NC = 2
NS = 16
NW = NC * NS
CHUNK = 128
NBUF = 4
EMB = 64


@functools.cache
def _build(n_seq: int, n_tok: int):
    n_units = n_tok * (n_seq // CHUNK)
    per_w = n_units // NW
    assert per_w * NW == n_units and per_w > NBUF
    sblocks = n_seq // CHUNK

    mesh = plsc.VectorSubcoreMesh(core_axis_name="c", subcore_axis_name="s")

    @functools.partial(
        pl.kernel,
        mesh=mesh,
        out_type=jax.ShapeDtypeStruct(
            (n_tok, EMB // 8, sblocks, 8, CHUNK, 1), jnp.float32),
        scratch_types=(
            [pltpu.VMEM((per_w, CHUNK), jnp.int32)]
            + [pltpu.VMEM((CHUNK, EMB), jnp.float32) for _ in range(NBUF)]
            + [pltpu.SemaphoreType.DMA for _ in range(2 * NBUF)]
        ),
        compiler_params=pltpu.CompilerParams(use_tc_tiling_on_sc=False,
                                             needs_layout_passes=False),
    )
    def emb(tok_hbm, w_hbm, out_hbm, idx_v, *rest):
        rows = rest[:NBUF]
        gsem = rest[NBUF:2 * NBUF]
        wsem = rest[2 * NBUF:]
        wid = lax.axis_index("s") * NC + lax.axis_index("c")
        u0 = wid * per_w
        pltpu.sync_copy(tok_hbm.at[wid], idx_v)

        def fire(c, m):
            pltpu.async_copy(w_hbm.at[idx_v.at[c]], rows[m], gsem[m])

        def unit_tsb(c):
            u = u0 + c
            return u // sblocks, u % sblocks

        def drain_gather(m):
            pltpu.make_async_copy(w_hbm.at[pl.ds(0, CHUNK)], rows[m],
                                  gsem[m]).wait()

        def start_write(c, m):
            t, sb = unit_tsb(c)

            def wbody(e, carry):
                pltpu.async_copy(rows[m].at[pl.ds(0, CHUNK), pl.ds(e, 1)],
                                 out_hbm.at[t, e // 8, sb, e % 8], wsem[m])
                return carry

            lax.fori_loop(0, EMB, wbody, 0)

        def drain_write(c, m):
            t, sb = unit_tsb(c)

            def dbody(e, carry):
                pltpu.make_async_copy(
                    rows[m].at[pl.ds(0, CHUNK), pl.ds(e, 1)],
                    out_hbm.at[t, e // 8, sb, e % 8], wsem[m]).wait()
                return carry

            lax.fori_loop(0, EMB, dbody, 0)

        for m in range(NBUF):
            fire(m, m)

        def body(j, carry):
            c0 = j * NBUF
            for m in range(NBUF):
                drain_gather(m)
                start_write(c0 + m, m)
            for m in range(NBUF):
                drain_write(c0 + m, m)
                fire(c0 + NBUF + m, m)
            return carry

        nout = per_w // NBUF
        lax.fori_loop(0, nout - 1, body, 0)

        c0 = (nout - 1) * NBUF
        for m in range(NBUF):
            drain_gather(m)
            start_write(c0 + m, m)
        for m in range(NBUF):
            drain_write(c0 + m, m)

    return emb


def kernel(token_ids, weight):
    s, t = token_ids.shape
    tok = token_ids.astype(jnp.int32).T.reshape(NW, (s // CHUNK) * t // NW,
                                                CHUNK)
    o6 = _build(s, t)(tok, weight)
    return o6.reshape(t, EMB // 8, s // CHUNK, 8, CHUNK).transpose(
        2, 4, 0, 1, 3).reshape(s, t, EMB)


# speedup vs baseline: 136.9774x; 136.9774x over previous
import functools

import jax
import jax.numpy as jnp
from jax import lax
from jax.experimental import pallas as pl
from jax.experimental.pallas import tpu as pltpu
from jax.experimental.pallas import tpu_sc as plsc

NC = 2
NS = 16
NW = NC * NS
CHUNK = 128
NBUF = 4
EMB = 64
LANES = 16


@functools.cache
def _build(n_seq: int, n_tok: int):
    n_units = n_tok * (n_seq // CHUNK)
    per_w = n_units // NW
    assert per_w * NW == n_units and per_w > NBUF
    sblocks = n_seq // CHUNK

    mesh = plsc.VectorSubcoreMesh(core_axis_name="c", subcore_axis_name="s")

    @functools.partial(
        pl.kernel,
        mesh=mesh,
        out_type=jax.ShapeDtypeStruct(
            (n_tok, EMB // 8, sblocks, 8, CHUNK), jnp.float32),
        scratch_types=(
            [pltpu.VMEM((per_w, CHUNK), jnp.int32)]
            + [pltpu.VMEM((CHUNK, EMB), jnp.float32) for _ in range(NBUF)]
            + [pltpu.VMEM((EMB // 8, 8, CHUNK + 1), jnp.float32)
               for _ in range(NBUF)]
            + [pltpu.SemaphoreType.DMA for _ in range(2 * NBUF)]
        ),
        compiler_params=pltpu.CompilerParams(use_tc_tiling_on_sc=False,
                                             needs_layout_passes=False),
    )
    def emb(tok_hbm, w_hbm, out_hbm, idx_v, *rest):
        rows = rest[:NBUF]
        tes = rest[NBUF:2 * NBUF]
        gsem = rest[2 * NBUF:3 * NBUF]
        wsem = rest[3 * NBUF:]
        wid = lax.axis_index("s") * NC + lax.axis_index("c")
        u0 = wid * per_w
        pltpu.sync_copy(tok_hbm.at[wid], idx_v)

        def fire(c, m):
            pltpu.async_copy(w_hbm.at[idx_v.at[c]], rows[m], gsem[m])

        def unit_tsb(c):
            u = u0 + c
            return u // sblocks, u % sblocks

        def drain_gather(c, m):
            pltpu.make_async_copy(w_hbm.at[pl.ds(0, CHUNK)], rows[m],
                                  gsem[m]).wait()

        lane = lax.broadcasted_iota(jnp.int32, (LANES,), 0)

        ebv = [(lane + evb * LANES) // 8 for evb in range(EMB // LANES)]
        eiv = [(lane + evb * LANES) % 8 for evb in range(EMB // LANES)]

        def transpose(m):
            def tbody(q, carry):
                for u in range(8):
                    si = q * 8 + u
                    vs = [rows[m][si, pl.ds(evb * LANES, LANES)]
                          for evb in range(EMB // LANES)]
                    siv = jnp.full((LANES,), si, jnp.int32)
                    for evb in range(EMB // LANES):
                        plsc.store_scatter(tes[m], [ebv[evb], eiv[evb], siv],
                                           vs[evb])
                return carry

            lax.fori_loop(0, CHUNK // 8, tbody, 0)

        def start_write(c, m):
            t, sb = unit_tsb(c)
            for eb in range(EMB // 8):
                pltpu.async_copy(tes[m].at[eb, :, pl.ds(0, CHUNK)],
                                 out_hbm.at[t, eb, sb], wsem[m])

        def drain_write(c, m):
            t, sb = unit_tsb(c)
            for eb in range(EMB // 8):
                pltpu.make_async_copy(tes[m].at[eb, :, pl.ds(0, CHUNK)],
                                      out_hbm.at[t, eb, sb], wsem[m]).wait()

        for m in range(NBUF):
            fire(m, m)

        def body(j, carry):
            c0 = j * NBUF
            for m in range(NBUF):
                drain_gather(c0 + m, m)

                @pl.when(j >= 1)
                def _():
                    drain_write(c0 + m - NBUF, m)

                transpose(m)
                start_write(c0 + m, m)
                fire(c0 + NBUF + m, m)
            return carry

        nout = per_w // NBUF
        lax.fori_loop(0, nout - 1, body, 0)

        c0 = (nout - 1) * NBUF
        for m in range(NBUF):
            drain_gather(c0 + m, m)
            drain_write(c0 + m - NBUF, m)
            transpose(m)
            start_write(c0 + m, m)
        for m in range(NBUF):
            drain_write(c0 + m, m)

    return emb


def kernel(token_ids, weight):
    s, t = token_ids.shape
    tok = token_ids.astype(jnp.int32).T.reshape(NW, (s // CHUNK) * t // NW,
                                                CHUNK)
    o5 = _build(s, t)(tok, weight)
    return o5.transpose(2, 4, 0, 1, 3).reshape(s, t, EMB)
